# Initial kernel scaffold; baseline (speedup 1.0000x reference)
import jax
import jax.numpy as jnp
from jax.experimental import pallas as pl


def kernel(volume, t_sorted, M, b, src, dst):
    raise NotImplementedError("write your pallas kernel here")


# speedup vs baseline: 1.0020x; 1.0020x over previous
import functools

import jax
import jax.numpy as jnp
from jax import lax
from jax.experimental import pallas as pl
from jax.experimental.pallas import tpu as pltpu
from jax.experimental.pallas import tpu_sc as plsc

_NC = 2
_NS = 16
_NW = _NC * _NS


def _geom_body(n_x, n_y, n_z, s_seg, t_ref, src_ref, dst_ref, minv_ref, b_ref,
               idx_ref, w_ref):
    t = t_ref[...]
    t0 = t[:, :s_seg]
    t1 = t[:, 1:]
    mids = []
    sq = None
    for d in range(3):
        s_d = src_ref[:, d][:, None]
        e_d = dst_ref[:, d][:, None]
        dd = e_d - s_d
        p0 = s_d + t0 * dd
        p1 = s_d + t1 * dd
        diff = p1 - p0
        sq = diff * diff if sq is None else sq + diff * diff
        mids.append(0.5 * (p0 + p1))
    seg_len = jnp.sqrt(sq)
    idx3 = []
    for r in range(3):
        acc = None
        for d in range(3):
            term = minv_ref[r, d] * (mids[d] - b_ref[d])
            acc = term if acc is None else acc + term
        idx3.append(jnp.round(acc).astype(jnp.int32))
    ii, jj, kk = idx3
    valid = ((ii >= 0) & (ii < n_x) & (jj >= 0) & (jj < n_y)
             & (kk >= 0) & (kk < n_z))
    flat = ii * (n_y * n_z) + jj * n_z + kk
    idx_ref[...] = jnp.where(valid, flat, 0)
    w_ref[...] = jnp.where(valid, seg_len, 0.0)


def _reduce_body(v_ref, w_ref, o_ref):
    o_ref[...] = jnp.sum(v_ref[...] * w_ref[...], axis=1, keepdims=True)


def kernel(volume, t_sorted, M, b, src, dst):
    n_x, n_y, n_z = volume.shape
    n_ray, k_t = t_sorted.shape
    s_seg = k_t - 1
    m_inv = jnp.linalg.inv(M)

    rows = 1024
    grid = (n_ray // rows,)
    idx, w = pl.pallas_call(
        functools.partial(_geom_body, n_x, n_y, n_z, s_seg),
        grid=grid,
        in_specs=[
            pl.BlockSpec((rows, k_t), lambda i: (i, 0)),
            pl.BlockSpec((rows, 3), lambda i: (i, 0)),
            pl.BlockSpec((rows, 3), lambda i: (i, 0)),
            pl.BlockSpec(memory_space=pltpu.SMEM),
            pl.BlockSpec(memory_space=pltpu.SMEM),
        ],
        out_specs=[
            pl.BlockSpec((rows, s_seg), lambda i: (i, 0)),
            pl.BlockSpec((rows, s_seg), lambda i: (i, 0)),
        ],
        out_shape=[
            jax.ShapeDtypeStruct((n_ray, s_seg), jnp.int32),
            jax.ShapeDtypeStruct((n_ray, s_seg), jnp.float32),
        ],
    )(t_sorted, src, dst, m_inv, b)

    n_idx = n_ray * s_seg
    per_w = n_idx // _NW
    chunk = 1024
    n_ch = per_w // chunk
    mesh = plsc.VectorSubcoreMesh(core_axis_name="c", subcore_axis_name="s")

    @functools.partial(
        pl.kernel,
        out_type=jax.ShapeDtypeStruct((n_idx,), jnp.float32),
        mesh=mesh,
        scratch_types=[
            pltpu.VMEM((chunk,), jnp.int32),
            pltpu.VMEM((chunk,), jnp.float32),
            pltpu.SemaphoreType.DMA,
        ],
    )
    def sc_gather(vol_hbm, idx_hbm, out_hbm, idx_v, val_v, sem):
        wid = lax.axis_index("s") * _NC + lax.axis_index("c")
        base = wid * per_w

        @pl.loop(0, n_ch)
        def _(c):
            off = base + c * chunk
            pltpu.sync_copy(idx_hbm.at[pl.ds(off, chunk)], idx_v)
            pltpu.async_copy(vol_hbm.at[idx_v], val_v, sem).wait()
            pltpu.sync_copy(val_v, out_hbm.at[pl.ds(off, chunk)])

    vals = sc_gather(volume.reshape(-1), idx.reshape(-1))

    rows2 = 2048
    out = pl.pallas_call(
        _reduce_body,
        grid=(n_ray // rows2,),
        in_specs=[
            pl.BlockSpec((rows2, s_seg), lambda i: (i, 0)),
            pl.BlockSpec((rows2, s_seg), lambda i: (i, 0)),
        ],
        out_specs=pl.BlockSpec((rows2, 1), lambda i: (i, 0)),
        out_shape=jax.ShapeDtypeStruct((n_ray, 1), jnp.float32),
    )(vals.reshape(n_ray, s_seg), w)
    return out.reshape(n_ray)
